# initial kernel scaffold (unmeasured)
import jax
import jax.numpy as jnp
from jax import lax
from jax.experimental import pallas as pl
from jax.experimental.pallas import tpu as pltpu

N_DEV = 32


def kernel(x, w_mat):
    m_per, k = x.shape
    _, n = w_mat.shape
    n_per = n // N_DEV
    m = m_per * N_DEV

    def body(x_ref, w_ref, out_ref,
             comm_send, comm_recv, amax_send, amax_recv,
             data_send_sems, data_recv_sems,
             amax_send_sems, amax_recv_sems):
        my = lax.axis_index("i")

        amax_recv[...] = jnp.zeros((N_DEV, 8, 128), jnp.float32)

        barrier_sem = pltpu.get_barrier_semaphore()
        for d in range(1, N_DEV):
            pl.semaphore_signal(
                barrier_sem, inc=1,
                device_id=((my + d) % N_DEV,),
                device_id_type=pl.DeviceIdType.MESH,
            )
        pl.semaphore_wait(barrier_sem, N_DEV - 1)

        y = lax.dot_general(
            x_ref[...], w_ref[...],
            (((1,), (0,)), ((), ())),
            preferred_element_type=jnp.float32,
            precision=lax.Precision.HIGHEST,
        )
        y = jnp.maximum(y, 0.0)
        local_amax = jnp.max(y)

        chunks = y.reshape(m_per, N_DEV, n_per).swapaxes(0, 1)
        comm_send[...] = chunks.reshape(N_DEV, m_per * n_per // 128, 128)
        amax_send[...] = jnp.full((8, 128), local_amax, jnp.float32)

        own = lax.dynamic_slice_in_dim(
            comm_send[...], my, 1, axis=0)
        pl.store(comm_recv,
                 (pl.dslice(my, 1), slice(None), slice(None)), own)

        data_rdmas = []
        amax_rdmas = []
        for d in range(1, N_DEV):
            p = (my + d) % N_DEV
            rd = pltpu.make_async_remote_copy(
                src_ref=comm_send.at[p],
                dst_ref=comm_recv.at[my],
                send_sem=data_send_sems.at[d],
                recv_sem=data_recv_sems.at[d],
                device_id=(p,),
                device_id_type=pl.DeviceIdType.MESH,
            )
            rd.start()
            data_rdmas.append(rd)
            ra = pltpu.make_async_remote_copy(
                src_ref=amax_send,
                dst_ref=amax_recv.at[my],
                send_sem=amax_send_sems.at[d],
                recv_sem=amax_recv_sems.at[d],
                device_id=(p,),
                device_id_type=pl.DeviceIdType.MESH,
            )
            ra.start()
            amax_rdmas.append(ra)

        for d in range(1, N_DEV):
            s = (my - d) % N_DEV
            recv_d = pltpu.make_async_remote_copy(
                src_ref=comm_send.at[0],
                dst_ref=comm_recv.at[s],
                send_sem=data_send_sems.at[0],
                recv_sem=data_recv_sems.at[d],
                device_id=(my,),
                device_id_type=pl.DeviceIdType.MESH,
            )
            recv_d.wait_recv()
            recv_a = pltpu.make_async_remote_copy(
                src_ref=amax_send,
                dst_ref=amax_recv.at[s],
                send_sem=amax_send_sems.at[0],
                recv_sem=amax_recv_sems.at[d],
                device_id=(my,),
                device_id_type=pl.DeviceIdType.MESH,
            )
            recv_a.wait_recv()

        g_amax = jnp.maximum(local_amax, jnp.max(amax_recv[...]))
        y_full = comm_recv[...].reshape(m, n_per)
        scale = g_amax / 127.0
        q = jnp.clip(jnp.round(y_full / scale), -127.0, 127.0)
        out_ref[...] = (q * scale).astype(jnp.float32)

        for rd in data_rdmas:
            rd.wait_send()
        for ra in amax_rdmas:
            ra.wait_send()

    return pl.pallas_call(
        body,
        out_shape=jax.ShapeDtypeStruct((m, n_per), jnp.float32),
        in_specs=[
            pl.BlockSpec(memory_space=pltpu.VMEM),
            pl.BlockSpec(memory_space=pltpu.VMEM),
        ],
        out_specs=pl.BlockSpec(memory_space=pltpu.VMEM),
        scratch_shapes=[
            pltpu.VMEM((N_DEV, m_per * n_per // 128, 128), jnp.float32),
            pltpu.VMEM((N_DEV, m_per * n_per // 128, 128), jnp.float32),
            pltpu.VMEM((8, 128), jnp.float32),
            pltpu.VMEM((N_DEV, 8, 128), jnp.float32),
            pltpu.SemaphoreType.DMA((N_DEV,)),
            pltpu.SemaphoreType.DMA((N_DEV,)),
            pltpu.SemaphoreType.DMA((N_DEV,)),
            pltpu.SemaphoreType.DMA((N_DEV,)),
        ],
        compiler_params=pltpu.CompilerParams(collective_id=0),
    )(x, w_mat)


# baseline (device time: 68039 ns/iter reference)
import jax
import jax.numpy as jnp
from jax import lax
from jax.experimental import pallas as pl
from jax.experimental.pallas import tpu as pltpu

N_DEV = 32


def kernel(x, w_mat):
    m_per, k = x.shape
    _, n = w_mat.shape
    n_per = n // N_DEV
    m = m_per * N_DEV

    def body(x_ref, w_ref, out_ref,
             comm_send, comm_recv, amax_send, amax_recv,
             data_send_sems, data_recv_sems,
             amax_send_sems, amax_recv_sems):
        my = lax.axis_index("i")

        amax_recv[...] = jnp.zeros((N_DEV, 8, 128), jnp.float32)

        barrier_sem = pltpu.get_barrier_semaphore()
        for d in range(1, N_DEV):
            pl.semaphore_signal(
                barrier_sem, inc=1,
                device_id=((my + d) % N_DEV,),
                device_id_type=pl.DeviceIdType.MESH,
            )
        pl.semaphore_wait(barrier_sem, N_DEV - 1)

        y = lax.dot_general(
            x_ref[...], w_ref[...],
            (((1,), (0,)), ((), ())),
            preferred_element_type=jnp.float32,
            precision=lax.Precision.HIGHEST,
        )
        y = jnp.maximum(y, 0.0)
        local_amax = jnp.max(y)

        comm_send[...] = y.reshape(m_per, N_DEV, n_per).swapaxes(0, 1)
        amax_send[...] = jnp.full((8, 128), local_amax, jnp.float32)

        comm_recv[pl.ds(my, 1), :, :] = comm_send[pl.ds(my, 1), :, :]

        data_rdmas = []
        amax_rdmas = []
        for d in range(1, N_DEV):
            p = (my + d) % N_DEV
            rd = pltpu.make_async_remote_copy(
                src_ref=comm_send.at[p],
                dst_ref=comm_recv.at[my],
                send_sem=data_send_sems.at[d],
                recv_sem=data_recv_sems.at[d],
                device_id=(p,),
                device_id_type=pl.DeviceIdType.MESH,
            )
            rd.start()
            data_rdmas.append(rd)
            ra = pltpu.make_async_remote_copy(
                src_ref=amax_send,
                dst_ref=amax_recv.at[my],
                send_sem=amax_send_sems.at[d],
                recv_sem=amax_recv_sems.at[d],
                device_id=(p,),
                device_id_type=pl.DeviceIdType.MESH,
            )
            ra.start()
            amax_rdmas.append(ra)

        for d in range(1, N_DEV):
            s = (my - d) % N_DEV
            recv_d = pltpu.make_async_remote_copy(
                src_ref=comm_send.at[0],
                dst_ref=comm_recv.at[s],
                send_sem=data_send_sems.at[0],
                recv_sem=data_recv_sems.at[d],
                device_id=(my,),
                device_id_type=pl.DeviceIdType.MESH,
            )
            recv_d.wait_recv()
            recv_a = pltpu.make_async_remote_copy(
                src_ref=amax_send,
                dst_ref=amax_recv.at[s],
                send_sem=amax_send_sems.at[0],
                recv_sem=amax_recv_sems.at[d],
                device_id=(my,),
                device_id_type=pl.DeviceIdType.MESH,
            )
            recv_a.wait_recv()

        g_amax = jnp.maximum(local_amax, jnp.max(amax_recv[...]))
        y_full = comm_recv[...].reshape(m, n_per)
        scale = g_amax / 127.0
        q = jnp.clip(jnp.round(y_full / scale), -127.0, 127.0)
        out_ref[...] = (q * scale).astype(jnp.float32)

        for rd in data_rdmas:
            rd.wait_send()
        for ra in amax_rdmas:
            ra.wait_send()

    return pl.pallas_call(
        body,
        out_shape=jax.ShapeDtypeStruct((m, n_per), jnp.float32),
        in_specs=[
            pl.BlockSpec(memory_space=pltpu.VMEM),
            pl.BlockSpec(memory_space=pltpu.VMEM),
        ],
        out_specs=pl.BlockSpec(memory_space=pltpu.VMEM),
        scratch_shapes=[
            pltpu.VMEM((N_DEV, m_per, n_per), jnp.float32),
            pltpu.VMEM((N_DEV, m_per, n_per), jnp.float32),
            pltpu.VMEM((8, 128), jnp.float32),
            pltpu.VMEM((N_DEV, 8, 128), jnp.float32),
            pltpu.SemaphoreType.DMA((N_DEV,)),
            pltpu.SemaphoreType.DMA((N_DEV,)),
            pltpu.SemaphoreType.DMA((N_DEV,)),
            pltpu.SemaphoreType.DMA((N_DEV,)),
        ],
        compiler_params=pltpu.CompilerParams(
            collective_id=0,
            vmem_limit_bytes=100 * 1024 * 1024,
        ),
    )(x, w_mat)


# device time: 59470 ns/iter; 1.1441x vs baseline; 1.1441x over previous
import jax
import jax.numpy as jnp
from jax import lax
from jax.experimental import pallas as pl
from jax.experimental.pallas import tpu as pltpu

N_DEV = 32
G = 8


def kernel(x, w_mat):
    m_per, k = x.shape
    _, n = w_mat.shape
    n_per = n // N_DEV
    m = m_per * N_DEV
    S = N_DEV // G
    cg = S * n_per

    def body(x_ref, w_ref, out_ref, comm_send, comm_recv,
             amax_send, amax_recv,
             data_send_sems, data_recv_sems,
             amax_send_sems, amax_recv_sems):
        my = lax.axis_index("i")

        amax_recv[...] = jnp.zeros((N_DEV, 8, 128), jnp.float32)

        barrier_sem = pltpu.get_barrier_semaphore()
        for d in range(1, N_DEV):
            pl.semaphore_signal(
                barrier_sem, inc=1,
                device_id=((my + d) % N_DEV,),
                device_id_type=pl.DeviceIdType.MESH,
            )

        x_val = x_ref[...]
        local_amax = jnp.float32(0.0)
        data_rdmas = []
        for g in range(G):
            y_g = lax.dot_general(
                x_val, w_ref[:, g * cg:(g + 1) * cg],
                (((1,), (0,)), ((), ())),
                preferred_element_type=jnp.float32,
                precision=lax.Precision.DEFAULT,
            )
            y_g = jnp.maximum(y_g, 0.0)
            local_amax = jnp.maximum(local_amax, jnp.max(y_g))

            if g == 0:
                pl.semaphore_wait(barrier_sem, N_DEV - 1)

            for j in range(S):
                p = g * S + j
                chunk = y_g[:, j * n_per:(j + 1) * n_per]
                comm_send[p, :, :] = chunk
                rd = pltpu.make_async_remote_copy(
                    src_ref=comm_send.at[p],
                    dst_ref=comm_recv.at[my],
                    send_sem=data_send_sems.at[p],
                    recv_sem=data_recv_sems.at[my],
                    device_id=(p,),
                    device_id_type=pl.DeviceIdType.MESH,
                )

                @pl.when(p != my)
                def _():
                    rd.start()

                data_rdmas.append((p, rd))

                @pl.when(p == my)
                def _():
                    comm_recv[pl.ds(my, 1), :, :] = chunk[None, :, :]

        amax_send[...] = jnp.full((8, 128), local_amax, jnp.float32)
        amax_rdmas = []
        for p in range(N_DEV):
            ra = pltpu.make_async_remote_copy(
                src_ref=amax_send,
                dst_ref=amax_recv.at[my],
                send_sem=amax_send_sems.at[p],
                recv_sem=amax_recv_sems.at[my],
                device_id=(p,),
                device_id_type=pl.DeviceIdType.MESH,
            )

            @pl.when(p != my)
            def _():
                ra.start()

            amax_rdmas.append((p, ra))

        for s in range(N_DEV):
            recv_d = pltpu.make_async_remote_copy(
                src_ref=comm_send.at[s],
                dst_ref=comm_recv.at[s],
                send_sem=data_send_sems.at[0],
                recv_sem=data_recv_sems.at[s],
                device_id=(s,),
                device_id_type=pl.DeviceIdType.MESH,
            )
            recv_a = pltpu.make_async_remote_copy(
                src_ref=amax_send,
                dst_ref=amax_recv.at[s],
                send_sem=amax_send_sems.at[0],
                recv_sem=amax_recv_sems.at[s],
                device_id=(s,),
                device_id_type=pl.DeviceIdType.MESH,
            )

            @pl.when(s != my)
            def _():
                recv_d.wait_recv()
                recv_a.wait_recv()

        g_amax = jnp.maximum(local_amax, jnp.max(amax_recv[...]))
        y_full = comm_recv[...].reshape(m, n_per)
        scale = g_amax / 127.0
        q = jnp.clip(jnp.round(y_full / scale), -127.0, 127.0)
        out_ref[...] = (q * scale).astype(jnp.float32)

        for p, rd in data_rdmas:
            @pl.when(p != my)
            def _():
                rd.wait_send()
        for p, ra in amax_rdmas:
            @pl.when(p != my)
            def _():
                ra.wait_send()

    return pl.pallas_call(
        body,
        out_shape=jax.ShapeDtypeStruct((m, n_per), jnp.float32),
        in_specs=[
            pl.BlockSpec(memory_space=pltpu.VMEM),
            pl.BlockSpec(memory_space=pltpu.VMEM),
        ],
        out_specs=pl.BlockSpec(memory_space=pltpu.VMEM),
        scratch_shapes=[
            pltpu.VMEM((N_DEV, m_per, n_per), jnp.float32),
            pltpu.VMEM((N_DEV, m_per, n_per), jnp.float32),
            pltpu.VMEM((8, 128), jnp.float32),
            pltpu.VMEM((N_DEV, 8, 128), jnp.float32),
            pltpu.SemaphoreType.DMA((N_DEV,)),
            pltpu.SemaphoreType.DMA((N_DEV,)),
            pltpu.SemaphoreType.DMA((N_DEV,)),
            pltpu.SemaphoreType.DMA((N_DEV,)),
        ],
        compiler_params=pltpu.CompilerParams(
            collective_id=0,
            vmem_limit_bytes=100 * 1024 * 1024,
        ),
    )(x, w_mat)
